# baseline (speedup 1.0000x reference)
import jax
import jax.numpy as jnp
from jax.experimental import pallas as pl
from jax.experimental.pallas import tpu as pltpu

B, P, D, H = 32, 64, 2, 64
G = 8


def _body(na_ref, ea_ref, wn_ref, bn_ref, wet_ref, bet_ref,
          wme_ref, wms_ref, wmr_ref, bm_ref, wu1_ref, wu2_ref, bu_ref,
          wd0_ref, bd0_ref, wd1_ref, bd1_ref, out0_ref, out1_ref):
    f32 = jnp.float32
    na = na_ref[...].reshape(G * P, D * 5)
    hn = jax.nn.relu(jnp.dot(na, wn_ref[...],
                             preferred_element_type=f32) + bn_ref[...])
    a_term = jnp.dot(hn, wms_ref[...], preferred_element_type=f32)
    c_term = jnp.dot(hn, wmr_ref[...], preferred_element_type=f32) + bm_ref[...]
    hes = []
    for g in range(G):
        het = jax.nn.relu(jnp.dot(wet_ref[...], ea_ref[g],
                                  preferred_element_type=f32) + bet_ref[...])
        hes.append(jnp.transpose(het))
    he = jnp.concatenate(hes, axis=0)
    eh = jnp.dot(he, wme_ref[...], preferred_element_type=f32)
    eh4 = eh.reshape(G, P, P, D * H)
    t = jax.nn.relu(eh4 + a_term.reshape(G, 1, P, D * H)
                    + c_term.reshape(G, P, 1, D * H))
    agg = jnp.sum(t, axis=2).reshape(G * P, D * H)
    h2 = jax.nn.relu(jnp.dot(hn, wu1_ref[...], preferred_element_type=f32)
                     + jnp.dot(agg, wu2_ref[...], preferred_element_type=f32)
                     + bu_ref[...])
    out0_ref[...] = (jnp.dot(h2, wd0_ref[...], preferred_element_type=f32)
                     + bd0_ref[...]).reshape(G, P, D)
    out1_ref[...] = (jnp.dot(h2, wd1_ref[...], preferred_element_type=f32)
                     + bd1_ref[...]).reshape(G, P, D)


def _blkdiag(w):
    r, c = w.shape
    z = jnp.zeros((r, c), w.dtype)
    return jnp.concatenate(
        [jnp.concatenate([w, z], axis=1), jnp.concatenate([z, w], axis=1)],
        axis=0)


def kernel(dq1, dq2, dp1, dp2, m, t, dt, length, k,
           Wn_enc, bn_enc, We_enc, be_enc, W_msg, b_msg, W_upd, b_upd,
           W_dec, b_dec):
    del t, dt
    m_rep = jnp.tile(m, (1, 1, D))
    na = jnp.stack([dq1, dq2, dp1, dp2, m_rep], axis=-1).reshape(B, P, D * 5)
    ea = jnp.stack([length.reshape(B, P * P), k.reshape(B, P * P)], axis=1)
    wme, wms, wmr = W_msg[:H], W_msg[H:2 * H], W_msg[2 * H:]
    wu1, wu2 = W_upd[:H], W_upd[H:]
    two = lambda v: jnp.concatenate([v, v]).reshape(1, -1)

    per_b3 = lambda shape: pl.BlockSpec(shape, lambda b: (b, 0, 0))
    const2 = lambda shape: pl.BlockSpec(shape, lambda b: (0, 0))

    out = pl.pallas_call(
        _body,
        grid=(B // G,),
        in_specs=[
            per_b3((G, P, D * 5)),
            per_b3((G, 2, P * P)),
            const2((D * 5, D * H)),
            const2((1, D * H)),
            const2((H, 2)),
            const2((H, 1)),
            const2((H, D * H)),
            const2((D * H, D * H)),
            const2((D * H, D * H)),
            const2((1, D * H)),
            const2((D * H, D * H)),
            const2((D * H, D * H)),
            const2((1, D * H)),
            const2((D * H, D)),
            const2((1, D)),
            const2((D * H, D)),
            const2((1, D)),
        ],
        out_specs=[per_b3((G, P, D)), per_b3((G, P, D))],
        out_shape=[jax.ShapeDtypeStruct((B, P, D), jnp.float32),
                   jax.ShapeDtypeStruct((B, P, D), jnp.float32)],
        compiler_params=pltpu.CompilerParams(
            dimension_semantics=("arbitrary",)),
    )(na, ea, _blkdiag(Wn_enc), two(bn_enc), We_enc.T, be_enc.reshape(-1, 1),
      jnp.concatenate([wme, wme], axis=1), _blkdiag(wms), _blkdiag(wmr),
      two(b_msg), _blkdiag(wu1), _blkdiag(wu2), two(b_upd),
      _blkdiag(W_dec[:, 0:1]), jnp.full((1, D), b_dec[0]),
      _blkdiag(W_dec[:, 1:2]), jnp.full((1, D), b_dec[1]))
    return out


# speedup vs baseline: 1.0975x; 1.0975x over previous
import jax
import jax.numpy as jnp
from jax.experimental import pallas as pl
from jax.experimental.pallas import tpu as pltpu

B, P, D, H = 32, 64, 2, 64
G = 8


def _body(na_ref, ea_ref, wn_ref, bn_ref, wet_ref, bet_ref,
          wme_ref, wms_ref, wmr_ref, bm_ref, wu1_ref, wu2_ref, bu_ref,
          wd_ref, bd_ref, out_ref):
    f32 = jnp.float32
    na = na_ref[...].reshape(G * P, D * 5)
    hn = jax.nn.relu(jnp.dot(na, wn_ref[...],
                             preferred_element_type=f32) + bn_ref[...])
    a_term = jnp.dot(hn, wms_ref[...], preferred_element_type=f32)
    c_term = jnp.dot(hn, wmr_ref[...], preferred_element_type=f32) + bm_ref[...]
    hes = []
    for g in range(G):
        het = jax.nn.relu(jnp.dot(wet_ref[...], ea_ref[g],
                                  preferred_element_type=f32) + bet_ref[...])
        hes.append(jnp.transpose(het))
    he = jnp.concatenate(hes, axis=0)
    eh = jnp.dot(he, wme_ref[...], preferred_element_type=f32)
    eh4 = eh.reshape(G, P, P, D * H)
    t = jax.nn.relu(eh4 + a_term.reshape(G, 1, P, D * H)
                    + c_term.reshape(G, P, 1, D * H))
    agg = jnp.sum(t, axis=2).reshape(G * P, D * H)
    h2 = jax.nn.relu(jnp.dot(hn, wu1_ref[...], preferred_element_type=f32)
                     + jnp.dot(agg, wu2_ref[...], preferred_element_type=f32)
                     + bu_ref[...])
    o = jnp.dot(h2, wd_ref[...], preferred_element_type=f32) + bd_ref[...]
    out_ref[...] = o.reshape(G, P, D * 2)


def _blkdiag(w):
    r, c = w.shape
    z = jnp.zeros((r, c), w.dtype)
    return jnp.concatenate(
        [jnp.concatenate([w, z], axis=1), jnp.concatenate([z, w], axis=1)],
        axis=0)


def kernel(dq1, dq2, dp1, dp2, m, t, dt, length, k,
           Wn_enc, bn_enc, We_enc, be_enc, W_msg, b_msg, W_upd, b_upd,
           W_dec, b_dec):
    del t, dt
    m_rep = jnp.tile(m, (1, 1, D))
    na = jnp.stack([dq1, dq2, dp1, dp2, m_rep], axis=-1).reshape(B, P, D * 5)
    ea = jnp.stack([length.reshape(B, P * P), k.reshape(B, P * P)], axis=1)
    wme, wms, wmr = W_msg[:H], W_msg[H:2 * H], W_msg[2 * H:]
    wu1, wu2 = W_upd[:H], W_upd[H:]
    two = lambda v: jnp.concatenate([v, v]).reshape(1, -1)

    per_b3 = lambda shape: pl.BlockSpec(shape, lambda b: (b, 0, 0))
    const2 = lambda shape: pl.BlockSpec(shape, lambda b: (0, 0))

    out = pl.pallas_call(
        _body,
        grid=(B // G,),
        in_specs=[
            per_b3((G, P, D * 5)),
            per_b3((G, 2, P * P)),
            const2((D * 5, D * H)),
            const2((1, D * H)),
            const2((H, 2)),
            const2((H, 1)),
            const2((H, D * H)),
            const2((D * H, D * H)),
            const2((D * H, D * H)),
            const2((1, D * H)),
            const2((D * H, D * H)),
            const2((D * H, D * H)),
            const2((1, D * H)),
            const2((D * H, D * 2)),
            const2((1, D * 2)),
        ],
        out_specs=per_b3((G, P, D * 2)),
        out_shape=jax.ShapeDtypeStruct((B, P, D * 2), jnp.float32),
        compiler_params=pltpu.CompilerParams(
            dimension_semantics=("arbitrary",)),
    )(na, ea, _blkdiag(Wn_enc), two(bn_enc), We_enc.T, be_enc.reshape(-1, 1),
      jnp.concatenate([wme, wme], axis=1), _blkdiag(wms), _blkdiag(wmr),
      two(b_msg), _blkdiag(wu1), _blkdiag(wu2), two(b_upd),
      _blkdiag(W_dec), two(b_dec))

    r = out.reshape(B, P, D, 2)
    return r[..., 0], r[..., 1]
